# baseline (speedup 1.0000x reference)
import jax
import jax.numpy as jnp
from jax import lax
from jax.experimental import pallas as pl
from jax.experimental.pallas import tpu as pltpu
from jax.experimental.pallas import tpu_sc as plsc

N_NODES = 10000
N_OPS = 5000
MAX_ARITY = 64
CUT = 32
EMB = 128

NC = 2
NS = 16
NW = NC * NS
OPS_PAD = 5120
OPT = OPS_PAD // NW
NB = 2
BM = 512
A_IN = 40


def _logits_body(e_ref, w_ref, lg_ref, eb_ref):
    e = e_ref[...]
    lg_ref[...] = jnp.dot(e, w_ref[...], preferred_element_type=jnp.float32)
    eb_ref[...] = e.astype(jnp.bfloat16)


def _logits(embs, w_extra):
    return pl.pallas_call(
        _logits_body,
        out_shape=(
            jax.ShapeDtypeStruct((N_NODES, 1), jnp.float32),
            jax.ShapeDtypeStruct((N_NODES, EMB), jnp.bfloat16),
        ),
    )(embs, w_extra)


def _gather_body(e2_hbm, ci_hbm, lgt_hbm, a_hbm,
                 ci_v, lgt_v, rows_v, tab, *sems):
    gsems, osems = sems[:NB], sems[NB:]
    cid = lax.axis_index("c")
    sid = lax.axis_index("s")
    wid = sid * NC + cid
    start = wid * OPT
    @pl.when(sid == 0)
    def _():
        pltpu.sync_copy(e2_hbm, tab)

    pltpu.sync_copy(ci_hbm.at[pl.ds(start, OPT)], ci_v)
    pltpu.sync_copy(lgt_hbm, lgt_v)
    plsc.subcore_barrier()

    def issue_gather(p, b):
        pltpu.async_copy(tab.at[ci_v.at[p]], rows_v.at[b], gsems[b])

    def wait_gather(p, b):
        pltpu.make_async_copy(
            tab.at[ci_v.at[p]], rows_v.at[b], gsems[b]).wait()

    def issue_out(p, b):
        pltpu.async_copy(rows_v.at[b, pl.ds(0, A_IN)],
                         a_hbm.at[start + p], osems[b])

    def wait_out(p, b):
        pltpu.make_async_copy(rows_v.at[b, pl.ds(0, A_IN)],
                              a_hbm.at[start + p], osems[b]).wait()

    for b in range(NB - 1):
        issue_gather(b, b)

    def step(g, carry):
        for b in range(NB):
            p = g * NB + b
            wait_gather(p, b)
            q = p + NB - 1
            bq = (b + NB - 1) % NB

            @pl.when(q < OPT)
            def _():
                @pl.when(p >= 1)
                def _():
                    wait_out(p - 1, bq)
                issue_gather(q, bq)
            c1 = ci_v[p, pl.ds(CUT, 16)]
            c2 = ci_v[p, pl.ds(CUT + 16, 16)]
            l1 = plsc.load_gather(lgt_v, [c1])
            l2 = plsc.load_gather(lgt_v, [c2])
            m = jnp.max(jnp.maximum(l1, l2))
            e1 = jnp.exp(l1 - m)
            e2 = jnp.exp(l2 - m)
            s = jnp.sum(e1 + e2)
            w1 = e1 / s
            w2 = e2 / s
            acc = [jnp.zeros((16,), jnp.float32) for _ in range(8)]
            for k in range(CUT):
                wk = w1[k] if k < 16 else w2[k - 16]
                for r in range(4):
                    pair = plsc.bitcast(rows_v[b, CUT + k, pl.ds(r * 16, 16)],
                                        jnp.bfloat16)
                    ua, ub = plsc.unpack(pair,
                                         format=plsc.PackFormat.INTERLEAVED)
                    acc[2 * r] = acc[2 * r] + wk * ua
                    acc[2 * r + 1] = acc[2 * r + 1] + wk * ub
            for r in range(4):
                rows_v[b, CUT, pl.ds(r * 16, 16)] = plsc.bitcast(
                    plsc.pack(acc[2 * r], acc[2 * r + 1],
                              format=plsc.PackFormat.INTERLEAVED),
                    jnp.int32)
            issue_out(p, b)
        return carry

    lax.fori_loop(0, OPT // NB, step, 0)
    for j in range(NB):
        p = OPT - NB + j
        wait_out(p, p % NB)


def _gather_aggregate(e2, ci_pad, lgt):
    mesh = plsc.VectorSubcoreMesh(core_axis_name="c", subcore_axis_name="s",
                                  num_cores=NC, num_subcores=NS)
    f = pl.kernel(
        _gather_body,
        out_type=jax.ShapeDtypeStruct((OPS_PAD, A_IN, EMB // 2), jnp.int32),
        mesh=mesh,
        compiler_params=pltpu.CompilerParams(needs_layout_passes=False),
        scratch_types=[
            pltpu.VMEM((OPT, MAX_ARITY), jnp.int32),
            pltpu.VMEM((N_NODES,), jnp.float32),
            pltpu.VMEM((NB, MAX_ARITY, EMB // 2), jnp.int32),
            pltpu.VMEM_SHARED((N_NODES, EMB // 2), jnp.int32),
        ] + [pltpu.SemaphoreType.DMA] * (2 * NB),
    )
    return f(e2, ci_pad, lgt)


def _cell_body(scat_ref, a_ref, w_ref, b_ref, e_ref, o_ref, res_ref):
    i = pl.program_id(0)

    @pl.when(i == 0)
    def _():
        o_ref[...] = e_ref[...]

    acc = b_ref[...]
    for j in range(CUT + 1):
        wb = w_ref[pl.ds(j * EMB, EMB), :].astype(jnp.bfloat16)
        acc = acc + jnp.dot(a_ref[:, j, :], wb,
                            preferred_element_type=jnp.float32)
    res_ref[...] = jnp.tanh(acc)

    base = i * BM
    nfull = (N_OPS // BM) * BM
    rem = N_OPS - nfull

    def body(p, carry):
        r = scat_ref[base + p]
        o_ref[pl.ds(r, 1), :] = res_ref[pl.ds(p, 1), :]
        return carry

    @pl.when(base < nfull)
    def _():
        lax.fori_loop(0, BM, body, 0, unroll=8)

    @pl.when(base == nfull)
    def _():
        lax.fori_loop(0, rem, body, 0, unroll=8)


def _cell_scatter(op_idx, a_mat, w_c, b_c, embs):
    grid_spec = pltpu.PrefetchScalarGridSpec(
        num_scalar_prefetch=1,
        grid=(OPS_PAD // BM,),
        in_specs=[
            pl.BlockSpec((BM, A_IN, EMB), lambda i, s: (i, 0, 0)),
            pl.BlockSpec(((CUT + 1) * EMB, EMB), lambda i, s: (0, 0)),
            pl.BlockSpec((1, EMB), lambda i, s: (0, 0)),
            pl.BlockSpec((N_NODES, EMB), lambda i, s: (0, 0)),
        ],
        out_specs=pl.BlockSpec((N_NODES, EMB), lambda i, s: (0, 0)),
        scratch_shapes=[pltpu.VMEM((BM, EMB), jnp.float32)],
    )
    return pl.pallas_call(
        _cell_body,
        grid_spec=grid_spec,
        out_shape=jax.ShapeDtypeStruct((N_NODES, EMB), jnp.float32),
    )(op_idx, a_mat, w_c, b_c, embs)


def kernel(embs, child_idx, op_idx, W_c, b_c, W_extra, b_extra):
    del b_extra
    ci_pad = jnp.zeros((OPS_PAD, MAX_ARITY), jnp.int32).at[:N_OPS].set(child_idx)
    lgt, ebf = _logits(embs, W_extra)
    e2 = jax.lax.bitcast_convert_type(
        ebf.reshape(N_NODES, EMB // 2, 2), jnp.int32)
    a_i32 = _gather_aggregate(e2, ci_pad, lgt.reshape(N_NODES))
    a_bf = jax.lax.bitcast_convert_type(
        a_i32, jnp.bfloat16).reshape(OPS_PAD, A_IN, EMB)
    return _cell_scatter(op_idx, a_bf, W_c, b_c.reshape(1, EMB), embs)


# speedup vs baseline: 3.2212x; 3.2212x over previous
import jax
import jax.numpy as jnp
from jax import lax
from jax.experimental import pallas as pl
from jax.experimental.pallas import tpu as pltpu
from jax.experimental.pallas import tpu_sc as plsc

N_NODES = 10000
N_OPS = 5000
MAX_ARITY = 64
CUT = 32
EMB = 128

NC = 2
NS = 16
NW = NC * NS
OPS_PAD = 5120
OPT = OPS_PAD // NW
NB = 2
BM = 512


def _logits_body(e_ref, w_ref, o_ref):
    o_ref[...] = jnp.dot(e_ref[...], w_ref[...],
                         preferred_element_type=jnp.float32)


def _logits(embs, w_extra):
    return pl.pallas_call(
        _logits_body,
        out_shape=jax.ShapeDtypeStruct((N_NODES, 1), jnp.float32),
    )(embs, w_extra)


def _gather_body(embs_hbm, ci_hbm, lgt_hbm, a_hbm,
                 ci_v, lgt_v, rows_v, tab, *sems):
    gsems, osems = sems[:NB], sems[NB:]
    cid = lax.axis_index("c")
    sid = lax.axis_index("s")
    wid = sid * NC + cid
    start = wid * OPT
    @pl.when(sid == 0)
    def _():
        pltpu.sync_copy(embs_hbm, tab)

    pltpu.sync_copy(ci_hbm.at[pl.ds(start, OPT)], ci_v)
    pltpu.sync_copy(lgt_hbm, lgt_v)
    plsc.subcore_barrier()

    def issue_gather(p, b):
        pltpu.async_copy(tab.at[ci_v.at[p]], rows_v.at[b], gsems[b])

    def wait_gather(p, b):
        pltpu.make_async_copy(
            tab.at[ci_v.at[p]], rows_v.at[b], gsems[b]).wait()

    def issue_out(p, b):
        pltpu.async_copy(rows_v.at[b, pl.ds(0, CUT + 1)],
                         a_hbm.at[start + p], osems[b])

    def wait_out(p, b):
        pltpu.make_async_copy(rows_v.at[b, pl.ds(0, CUT + 1)],
                              a_hbm.at[start + p], osems[b]).wait()

    for b in range(NB - 1):
        issue_gather(b, b)

    def step(g, carry):
        for b in range(NB):
            p = g * NB + b
            wait_gather(p, b)
            q = p + NB - 1
            bq = (b + NB - 1) % NB

            @pl.when(q < OPT)
            def _():
                @pl.when(p >= 1)
                def _():
                    wait_out(p - 1, bq)
                issue_gather(q, bq)
            c1 = ci_v[p, pl.ds(CUT, 16)]
            c2 = ci_v[p, pl.ds(CUT + 16, 16)]
            l1 = plsc.load_gather(lgt_v, [c1])
            l2 = plsc.load_gather(lgt_v, [c2])
            m = jnp.max(jnp.maximum(l1, l2))
            e1 = jnp.exp(l1 - m)
            e2 = jnp.exp(l2 - m)
            s = jnp.sum(e1 + e2)
            w1 = e1 / s
            w2 = e2 / s
            acc = [jnp.zeros((16,), jnp.float32) for _ in range(8)]
            for k in range(CUT):
                wk = w1[k] if k < 16 else w2[k - 16]
                for r in range(8):
                    acc[r] = acc[r] + wk * rows_v[b, CUT + k, pl.ds(r * 16, 16)]
            for r in range(8):
                rows_v[b, CUT, pl.ds(r * 16, 16)] = acc[r]
            issue_out(p, b)
        return carry

    lax.fori_loop(0, OPT // NB, step, 0)
    for j in range(NB):
        p = OPT - NB + j
        wait_out(p, p % NB)


def _gather_aggregate(embs, ci_pad, lgt):
    mesh = plsc.VectorSubcoreMesh(core_axis_name="c", subcore_axis_name="s",
                                  num_cores=NC, num_subcores=NS)
    f = pl.kernel(
        _gather_body,
        out_type=jax.ShapeDtypeStruct((OPS_PAD, CUT + 1, EMB), jnp.float32),
        mesh=mesh,
        compiler_params=pltpu.CompilerParams(needs_layout_passes=False),
        scratch_types=[
            pltpu.VMEM((OPT, MAX_ARITY), jnp.int32),
            pltpu.VMEM((N_NODES,), jnp.float32),
            pltpu.VMEM((NB, MAX_ARITY, EMB), jnp.float32),
            pltpu.VMEM_SHARED((N_NODES, EMB), jnp.float32),
        ] + [pltpu.SemaphoreType.DMA] * (2 * NB),
    )
    return f(embs, ci_pad, lgt)


def _cell_body(scat_ref, a_ref, w_ref, b_ref, e_ref, o_ref, res_ref):
    i = pl.program_id(0)

    @pl.when(i == 0)
    def _():
        o_ref[...] = e_ref[...]

    acc = b_ref[...]
    for j in range(CUT + 1):
        acc = acc + jnp.dot(a_ref[:, j, :], w_ref[pl.ds(j * EMB, EMB), :],
                            preferred_element_type=jnp.float32)
    res_ref[...] = jnp.tanh(acc)

    base = i * BM
    nfull = (N_OPS // BM) * BM
    rem = N_OPS - nfull

    def body(p, carry):
        r = scat_ref[base + p]
        o_ref[pl.ds(r, 1), :] = res_ref[pl.ds(p, 1), :]
        return carry

    @pl.when(base < nfull)
    def _():
        lax.fori_loop(0, BM, body, 0, unroll=8)

    @pl.when(base == nfull)
    def _():
        lax.fori_loop(0, rem, body, 0, unroll=8)


def _cell_scatter(op_idx, a_mat, w_c, b_c, embs):
    grid_spec = pltpu.PrefetchScalarGridSpec(
        num_scalar_prefetch=1,
        grid=(OPS_PAD // BM,),
        in_specs=[
            pl.BlockSpec((BM, CUT + 1, EMB), lambda i, s: (i, 0, 0)),
            pl.BlockSpec(((CUT + 1) * EMB, EMB), lambda i, s: (0, 0)),
            pl.BlockSpec((1, EMB), lambda i, s: (0, 0)),
            pl.BlockSpec((N_NODES, EMB), lambda i, s: (0, 0)),
        ],
        out_specs=pl.BlockSpec((N_NODES, EMB), lambda i, s: (0, 0)),
        scratch_shapes=[pltpu.VMEM((BM, EMB), jnp.float32)],
    )
    return pl.pallas_call(
        _cell_body,
        grid_spec=grid_spec,
        out_shape=jax.ShapeDtypeStruct((N_NODES, EMB), jnp.float32),
    )(op_idx, a_mat, w_c, b_c, embs)


def kernel(embs, child_idx, op_idx, W_c, b_c, W_extra, b_extra):
    del b_extra
    ci_pad = jnp.zeros((OPS_PAD, MAX_ARITY), jnp.int32).at[:N_OPS].set(child_idx)
    lgt = _logits(embs, W_extra).reshape(N_NODES)
    a_mat = _gather_aggregate(embs, ci_pad, lgt)
    return _cell_scatter(op_idx, a_mat, W_c, b_c.reshape(1, EMB), embs)
